# baseline (speedup 1.0000x reference)
import jax
import jax.numpy as jnp
from jax.experimental import pallas as pl
from jax.experimental.pallas import tpu as pltpu

_N_LABELS = 4
_BLOCK_ROWS = 16000


def _ggnn_body(e_ref, x_ref, w_ref, o_ref):
    x = x_ref[...]
    eb = e_ref[0]
    b, n = eb.shape
    ecol = eb[:, :, None]
    ps = []
    for i in range(_N_LABELS):
        p = jax.lax.dot_general(
            x, w_ref[i],
            dimension_numbers=(((1,), (0,)), ((), ())),
            preferred_element_type=jnp.float32,
        )
        ps.append(p.reshape(b, n, p.shape[1]))
    out = jnp.where(
        ecol < 2,
        jnp.where(ecol == 0, ps[0], ps[1]),
        jnp.where(ecol == 2, ps[2], ps[3]),
    )
    o_ref[...] = out.reshape(x.shape)


def kernel(h_v, h_w, e_vw, edge_matrix):
    del h_v
    nb, nn, nin = h_w.shape
    nout = edge_matrix.shape[1]
    rows = nb * nn
    br = _BLOCK_ROWS
    grid = (rows // br,)
    em_t = jnp.transpose(edge_matrix, (0, 2, 1))
    x2 = jnp.reshape(h_w, (rows, nin))
    e2 = jnp.reshape(e_vw, (rows // br, br // nn, nn))
    out = pl.pallas_call(
        _ggnn_body,
        grid=grid,
        in_specs=[
            pl.BlockSpec((1, br // nn, nn), lambda g: (g, 0, 0)),
            pl.BlockSpec((br, nin), lambda g: (g, 0)),
            pl.BlockSpec((_N_LABELS, nin, nout), lambda g: (0, 0, 0)),
        ],
        out_specs=pl.BlockSpec((br, nout), lambda g: (g, 0)),
        out_shape=jax.ShapeDtypeStruct((rows, nout), jnp.float32),
        compiler_params=pltpu.CompilerParams(
            dimension_semantics=("arbitrary",),
        ),
    )(e2, x2, em_t)
    return jnp.reshape(out, (nb, nn, nout))


# speedup vs baseline: 1.0004x; 1.0004x over previous
import jax
import jax.numpy as jnp
from jax.experimental import pallas as pl
from jax.experimental.pallas import tpu as pltpu

_N_LABELS = 4
_BLOCK_ROWS = 16000


def _ggnn_body(e_ref, x_ref, w_ref, o_ref):
    x = x_ref[...]
    xb = x.astype(jnp.bfloat16)
    eb = e_ref[0]
    b, n = eb.shape
    ecol = eb[:, :, None]
    ps = []
    for i in range(_N_LABELS):
        p = jax.lax.dot_general(
            xb, w_ref[i],
            dimension_numbers=(((1,), (0,)), ((), ())),
            preferred_element_type=jnp.float32,
        )
        ps.append(p.reshape(b, n, p.shape[1]))
    out = jnp.where(
        ecol < 2,
        jnp.where(ecol == 0, ps[0], ps[1]),
        jnp.where(ecol == 2, ps[2], ps[3]),
    )
    o_ref[...] = out.reshape(x.shape)


def kernel(h_v, h_w, e_vw, edge_matrix):
    del h_v
    nb, nn, nin = h_w.shape
    nout = edge_matrix.shape[1]
    rows = nb * nn
    br = _BLOCK_ROWS
    grid = (rows // br,)
    em_t = jnp.transpose(edge_matrix, (0, 2, 1)).astype(jnp.bfloat16)
    x2 = jnp.reshape(h_w, (rows, nin))
    e2 = jnp.reshape(e_vw, (rows // br, br // nn, nn))
    out = pl.pallas_call(
        _ggnn_body,
        grid=grid,
        in_specs=[
            pl.BlockSpec((1, br // nn, nn), lambda g: (g, 0, 0)),
            pl.BlockSpec((br, nin), lambda g: (g, 0)),
            pl.BlockSpec((_N_LABELS, nin, nout), lambda g: (0, 0, 0)),
        ],
        out_specs=pl.BlockSpec((br, nout), lambda g: (g, 0)),
        out_shape=jax.ShapeDtypeStruct((rows, nout), jnp.float32),
        compiler_params=pltpu.CompilerParams(
            dimension_semantics=("arbitrary",),
        ),
    )(e2, x2, em_t)
    return jnp.reshape(out, (nb, nn, nout))


# speedup vs baseline: 1.0008x; 1.0005x over previous
import jax
import jax.numpy as jnp
from jax.experimental import pallas as pl
from jax.experimental.pallas import tpu as pltpu

_N_LABELS = 4
_BLOCK_ROWS = 16000


def _ggnn_body(e_ref, x_ref, w_ref, o_ref):
    x = x_ref[...]
    eb = e_ref[0]
    b, n = eb.shape
    ecol = eb[:, :, None]
    x3 = x.astype(jnp.bfloat16).reshape(b, n, x.shape[1])
    zero = jnp.zeros((), jnp.bfloat16)
    x4 = jnp.concatenate(
        [jnp.where(ecol == i, x3, zero) for i in range(_N_LABELS)], axis=-1,
    ).reshape(x.shape[0], _N_LABELS * x.shape[1])
    o_ref[...] = jax.lax.dot_general(
        x4, w_ref[...],
        dimension_numbers=(((1,), (0,)), ((), ())),
        preferred_element_type=jnp.float32,
    )


def kernel(h_v, h_w, e_vw, edge_matrix):
    del h_v
    nb, nn, nin = h_w.shape
    nout = edge_matrix.shape[1]
    rows = nb * nn
    br = _BLOCK_ROWS
    grid = (rows // br,)
    em_t = jnp.transpose(edge_matrix, (0, 2, 1)).astype(jnp.bfloat16)
    em_s = jnp.reshape(em_t, (_N_LABELS * nin, nout))
    x2 = jnp.reshape(h_w, (rows, nin))
    e2 = jnp.reshape(e_vw, (rows // br, br // nn, nn))
    out = pl.pallas_call(
        _ggnn_body,
        grid=grid,
        in_specs=[
            pl.BlockSpec((1, br // nn, nn), lambda g: (g, 0, 0)),
            pl.BlockSpec((br, nin), lambda g: (g, 0)),
            pl.BlockSpec((_N_LABELS * nin, nout), lambda g: (0, 0)),
        ],
        out_specs=pl.BlockSpec((br, nout), lambda g: (g, 0)),
        out_shape=jax.ShapeDtypeStruct((rows, nout), jnp.float32),
        compiler_params=pltpu.CompilerParams(
            dimension_semantics=("arbitrary",),
        ),
    )(e2, x2, em_s)
    return jnp.reshape(out, (nb, nn, nout))
